# baseline (speedup 1.0000x reference)
import functools
import math

import jax
import jax.numpy as jnp
from jax.experimental import pallas as pl
from jax.experimental.pallas import tpu as pltpu

_NUM_HEADS = 16
_HEAD_SIZE = 128
_TOPK_RATIO = 0.1

def _attn_kernel(q_ref, k_ref, v_ref, o_ref, *, k_keep, scale):
    q = q_ref[...]
    k = k_ref[...]
    v = v_ref[...]

    st = jax.lax.dot_general(
        k, q, (((1,), (1,)), ((), ())), preferred_element_type=jnp.float32
    ) * scale

    b = jax.lax.bitcast_convert_type(st, jnp.int32)
    int_min = jnp.int32(-(2**31))
    keys = jnp.where(b >= 0, b, int_min - b)

    hi16 = (keys >> 16).astype(jnp.int16)
    lo16 = ((keys & 0xFFFF) - 32768).astype(jnp.int16)

    def count_ge16(x16, m16):
        h = x16.shape[0]
        parts = [
            jnp.where(x16[j : j + 128] >= m16, jnp.int16(1), jnp.int16(0))
            for j in range(0, h, 128)
        ]
        while len(parts) > 1:
            parts = [
                parts[i] + parts[i + 1] if i + 1 < len(parts) else parts[i]
                for i in range(0, len(parts), 2)
            ]
        acc = parts[0]
        while acc.shape[0] > 8:
            half = acc.shape[0] // 2
            acc = acc[:half] + acc[half:]
        return jnp.sum(acc.astype(jnp.int32), axis=0, keepdims=True)

    def body1(_, c):
        lo, hi = c
        mid = (lo >> 1) + (hi >> 1) + ((lo | hi) & 1)
        ge = count_ge16(hi16, mid.astype(jnp.int16)) >= k_keep
        return jnp.where(ge, mid, lo), jnp.where(ge, hi, mid - 1)

    qb = st.shape[1]
    i32row = lambda val: jnp.full((1, qb), val, dtype=jnp.int32)
    t_hi, _ = jax.lax.fori_loop(
        0, 16, body1, (i32row(-32768), i32row(32767))
    )
    t_hi16 = t_hi.astype(jnp.int16)

    c_gt = jnp.where(
        t_hi >= 32767, 0, count_ge16(hi16, (t_hi + 1).astype(jnp.int16))
    )
    k2 = k_keep - c_gt
    lo_m = jnp.where(hi16 == t_hi16, lo16, jnp.int16(-32768))

    def body2(_, c):
        lo, hi = c
        mid = (lo >> 1) + (hi >> 1) + ((lo | hi) & 1)
        ge = count_ge16(lo_m, mid.astype(jnp.int16)) >= k2
        return jnp.where(ge, mid, lo), jnp.where(ge, hi, mid - 1)

    t_lo, _ = jax.lax.fori_loop(
        0, 16, body2, (i32row(-32768), i32row(32767))
    )

    thresh = (t_hi << 16) + (t_lo + 32768)
    tb = jnp.where(thresh >= 0, thresh, int_min - thresh)
    thresh_f = jax.lax.bitcast_convert_type(tb, jnp.float32)
    mask = st >= thresh_f

    m = jnp.max(st, axis=0, keepdims=True)
    p = jnp.where(mask, jnp.exp(st - m), 0.0)
    denom = jnp.sum(p, axis=0, keepdims=True)
    pn = p * (1.0 / denom)
    out = jax.lax.dot_general(
        pn, v, (((0,), (0,)), ((), ())), preferred_element_type=jnp.float32
    )
    o_ref[...] = out


@jax.jit
def kernel(query, key, value):
    B, S, HD = query.shape
    H, D = _NUM_HEADS, _HEAD_SIZE
    assert B == 1 and HD == H * D
    k_keep = max(1, int(_TOPK_RATIO * S))
    scale = 1.0 / math.sqrt(D)

    q2 = query.reshape(S, HD)
    k2 = key.reshape(S, HD)
    v2 = value.reshape(S, HD)

    QB = 512
    grid = (H, S // QB)

    out = pl.pallas_call(
        functools.partial(_attn_kernel, k_keep=k_keep, scale=scale),
        grid=grid,
        in_specs=[
            pl.BlockSpec((QB, D), lambda h, qi: (qi, h)),
            pl.BlockSpec((S, D), lambda h, qi: (0, h)),
            pl.BlockSpec((S, D), lambda h, qi: (0, h)),
        ],
        out_specs=pl.BlockSpec((QB, D), lambda h, qi: (qi, h)),
        out_shape=jax.ShapeDtypeStruct((S, HD), jnp.float32),
        compiler_params=pltpu.CompilerParams(
            dimension_semantics=("parallel", "parallel"),
        ),
    )(q2, k2, v2)

    return out.reshape(B, S, HD)


# speedup vs baseline: 1.0688x; 1.0688x over previous
import functools
import math

import jax
import jax.numpy as jnp
from jax.experimental import pallas as pl
from jax.experimental.pallas import tpu as pltpu

_NUM_HEADS = 16
_HEAD_SIZE = 128
_TOPK_RATIO = 0.1

def _attn_kernel(q_ref, k_ref, v_ref, o_ref, *, k_keep, scale):
    q = q_ref[...]
    k = k_ref[...]
    v = v_ref[...]

    st = jax.lax.dot_general(
        k, q, (((1,), (1,)), ((), ())), preferred_element_type=jnp.float32
    ) * scale

    b = jax.lax.bitcast_convert_type(st, jnp.int32)
    int_min = jnp.int32(-(2**31))
    keys = jnp.where(b >= 0, b, int_min - b)

    hi16 = (keys >> 16).astype(jnp.int16)
    lo16 = ((keys & 0xFFFF) - 32768).astype(jnp.int16)

    def count_ge16(x16, m16):
        h = x16.shape[0]
        acc = None
        for j in range(0, h, 128):
            ind = jnp.where(x16[j : j + 128] >= m16, jnp.int16(1), jnp.int16(0))
            acc = ind if acc is None else acc + ind
        while acc.shape[0] > 8:
            half = acc.shape[0] // 2
            acc = acc[:half] + acc[half:]
        return jnp.sum(acc.astype(jnp.int32), axis=0, keepdims=True)

    def body1(_, c):
        lo, hi = c
        mid = (lo >> 1) + (hi >> 1) + ((lo | hi) & 1)
        ge = count_ge16(hi16, mid.astype(jnp.int16)) >= k_keep
        return jnp.where(ge, mid, lo), jnp.where(ge, hi, mid - 1)

    qb = st.shape[1]
    i32row = lambda val: jnp.full((1, qb), val, dtype=jnp.int32)
    t_hi, _ = jax.lax.fori_loop(
        0, 16, body1, (i32row(-32768), i32row(32767))
    )
    t_hi16 = t_hi.astype(jnp.int16)

    c_gt = jnp.where(
        t_hi >= 32767, 0, count_ge16(hi16, (t_hi + 1).astype(jnp.int16))
    )
    k2 = k_keep - c_gt
    lo_m = jnp.where(hi16 == t_hi16, lo16, jnp.int16(-32768))

    def body2(_, c):
        lo, hi = c
        mid = (lo >> 1) + (hi >> 1) + ((lo | hi) & 1)
        ge = count_ge16(lo_m, mid.astype(jnp.int16)) >= k2
        return jnp.where(ge, mid, lo), jnp.where(ge, hi, mid - 1)

    t_lo, _ = jax.lax.fori_loop(
        0, 16, body2, (i32row(-32768), i32row(32767))
    )

    thresh = (t_hi << 16) + (t_lo + 32768)
    tb = jnp.where(thresh >= 0, thresh, int_min - thresh)
    thresh_f = jax.lax.bitcast_convert_type(tb, jnp.float32)
    mask = st >= thresh_f

    m = jnp.max(st, axis=0, keepdims=True)
    p = jnp.where(mask, jnp.exp(st - m), 0.0)
    denom = jnp.sum(p, axis=0, keepdims=True)
    pn = p * (1.0 / denom)
    out = jax.lax.dot_general(
        pn, v, (((0,), (0,)), ((), ())), preferred_element_type=jnp.float32
    )
    o_ref[...] = out


@jax.jit
def kernel(query, key, value):
    B, S, HD = query.shape
    H, D = _NUM_HEADS, _HEAD_SIZE
    assert B == 1 and HD == H * D
    k_keep = max(1, int(_TOPK_RATIO * S))
    scale = 1.0 / math.sqrt(D)

    q2 = query.reshape(S, HD)
    k2 = key.reshape(S, HD)
    v2 = value.reshape(S, HD)

    QB = 512
    grid = (H, S // QB)

    out = pl.pallas_call(
        functools.partial(_attn_kernel, k_keep=k_keep, scale=scale),
        grid=grid,
        in_specs=[
            pl.BlockSpec((QB, D), lambda h, qi: (qi, h)),
            pl.BlockSpec((S, D), lambda h, qi: (0, h)),
            pl.BlockSpec((S, D), lambda h, qi: (0, h)),
        ],
        out_specs=pl.BlockSpec((QB, D), lambda h, qi: (qi, h)),
        out_shape=jax.ShapeDtypeStruct((S, HD), jnp.float32),
        compiler_params=pltpu.CompilerParams(
            dimension_semantics=("parallel", "parallel"),
        ),
    )(q2, k2, v2)

    return out.reshape(B, S, HD)


# speedup vs baseline: 1.1380x; 1.0648x over previous
import functools
import math

import jax
import jax.numpy as jnp
from jax.experimental import pallas as pl
from jax.experimental.pallas import tpu as pltpu

_NUM_HEADS = 16
_HEAD_SIZE = 128
_TOPK_RATIO = 0.1

def _attn_kernel(q_ref, k_ref, v_ref, o_ref, *, k_keep, scale):
    q = q_ref[...]
    k = k_ref[...]
    v = v_ref[...]

    st = jax.lax.dot_general(
        k, q, (((1,), (1,)), ((), ())), preferred_element_type=jnp.float32
    ) * scale

    b = jax.lax.bitcast_convert_type(st, jnp.int32)
    int_min = jnp.int32(-(2**31))
    keys = jnp.where(b >= 0, b, int_min - b)

    hi16 = (keys >> 16).astype(jnp.int16)
    lo16 = ((keys & 0xFFFF) - 32768).astype(jnp.int16)

    def count_ge16(x16, m16):
        h = x16.shape[0]
        acc = None
        for j in range(0, h, 128):
            ind = jnp.where(x16[j : j + 128] >= m16, jnp.int16(1), jnp.int16(0))
            acc = ind if acc is None else acc + ind
        while acc.shape[0] > 8:
            half = acc.shape[0] // 2
            acc = acc[:half] + acc[half:]
        return jnp.sum(acc.astype(jnp.int32), axis=0, keepdims=True)

    def body1(_, c):
        lo, hi = c
        mid = (lo >> 1) + (hi >> 1) + ((lo | hi) & 1)
        ge = count_ge16(hi16, mid.astype(jnp.int16)) >= k_keep
        return jnp.where(ge, mid, lo), jnp.where(ge, hi, mid - 1)

    qb = st.shape[1]
    i32row = lambda val: jnp.full((1, qb), val, dtype=jnp.int32)
    c = (i32row(-32768), i32row(32767))
    for _ in range(16):
        c = body1(None, c)
    t_hi = c[0]
    t_hi16 = t_hi.astype(jnp.int16)

    c_gt = jnp.where(
        t_hi >= 32767, 0, count_ge16(hi16, (t_hi + 1).astype(jnp.int16))
    )
    k2 = k_keep - c_gt
    lo_m = jnp.where(hi16 == t_hi16, lo16, jnp.int16(-32768))

    def body2(_, c):
        lo, hi = c
        mid = (lo >> 1) + (hi >> 1) + ((lo | hi) & 1)
        ge = count_ge16(lo_m, mid.astype(jnp.int16)) >= k2
        return jnp.where(ge, mid, lo), jnp.where(ge, hi, mid - 1)

    c = (i32row(-32768), i32row(32767))
    for _ in range(16):
        c = body2(None, c)
    t_lo = c[0]

    thresh = (t_hi << 16) + (t_lo + 32768)
    tb = jnp.where(thresh >= 0, thresh, int_min - thresh)
    thresh_f = jax.lax.bitcast_convert_type(tb, jnp.float32)
    mask = st >= thresh_f

    m = jnp.max(st, axis=0, keepdims=True)
    p = jnp.where(mask, jnp.exp(st - m), 0.0)
    denom = jnp.sum(p, axis=0, keepdims=True)
    pn = p * (1.0 / denom)
    out = jax.lax.dot_general(
        pn, v, (((0,), (0,)), ((), ())), preferred_element_type=jnp.float32
    )
    o_ref[...] = out


@jax.jit
def kernel(query, key, value):
    B, S, HD = query.shape
    H, D = _NUM_HEADS, _HEAD_SIZE
    assert B == 1 and HD == H * D
    k_keep = max(1, int(_TOPK_RATIO * S))
    scale = 1.0 / math.sqrt(D)

    q2 = query.reshape(S, HD)
    k2 = key.reshape(S, HD)
    v2 = value.reshape(S, HD)

    QB = 512
    grid = (H, S // QB)

    out = pl.pallas_call(
        functools.partial(_attn_kernel, k_keep=k_keep, scale=scale),
        grid=grid,
        in_specs=[
            pl.BlockSpec((QB, D), lambda h, qi: (qi, h)),
            pl.BlockSpec((S, D), lambda h, qi: (0, h)),
            pl.BlockSpec((S, D), lambda h, qi: (0, h)),
        ],
        out_specs=pl.BlockSpec((QB, D), lambda h, qi: (qi, h)),
        out_shape=jax.ShapeDtypeStruct((S, HD), jnp.float32),
        compiler_params=pltpu.CompilerParams(
            dimension_semantics=("parallel", "parallel"),
        ),
    )(q2, k2, v2)

    return out.reshape(B, S, HD)


# speedup vs baseline: 1.1925x; 1.0479x over previous
import functools
import math

import jax
import jax.numpy as jnp
from jax.experimental import pallas as pl
from jax.experimental.pallas import tpu as pltpu

_NUM_HEADS = 16
_HEAD_SIZE = 128
_TOPK_RATIO = 0.1

def _attn_kernel(q_ref, k_ref, v_ref, o_ref, *, k_keep, scale):
    q = q_ref[...]
    k = k_ref[...]
    v = v_ref[...]

    st = jax.lax.dot_general(
        k, q, (((1,), (1,)), ((), ())), preferred_element_type=jnp.float32
    ) * scale

    b = jax.lax.bitcast_convert_type(st, jnp.int32)
    int_min = jnp.int32(-(2**31))
    keys = jnp.where(b >= 0, b, int_min - b)

    hi16 = (keys >> 16).astype(jnp.int16)
    lo16 = ((keys & 0xFFFF) - 32768).astype(jnp.int16)

    def count_ge16(x16, m16):
        h = x16.shape[0]
        acc = None
        for j in range(0, h, 128):
            ind = jnp.where(x16[j : j + 128] >= m16, jnp.int16(1), jnp.int16(0))
            acc = ind if acc is None else acc + ind
        while acc.shape[0] > 8:
            half = acc.shape[0] // 2
            acc = acc[:half] + acc[half:]
        return jnp.sum(acc.astype(jnp.int32), axis=0, keepdims=True)

    def body1(_, c):
        lo, hi = c
        mid = (lo >> 1) + (hi >> 1) + ((lo | hi) & 1)
        ge = count_ge16(hi16, mid.astype(jnp.int16)) >= k_keep
        return jnp.where(ge, mid, lo), jnp.where(ge, hi, mid - 1)

    qb = st.shape[1]
    i32row = lambda val: jnp.full((1, qb), val, dtype=jnp.int32)
    c = (i32row(-32768), i32row(32767))
    for _ in range(16):
        c = body1(None, c)
    t_hi = c[0]
    t_hi16 = t_hi.astype(jnp.int16)

    c_gt = jnp.where(
        t_hi >= 32767, 0, count_ge16(hi16, (t_hi + 1).astype(jnp.int16))
    )
    k2 = k_keep - c_gt
    lo_m = jnp.where(hi16 == t_hi16, lo16, jnp.int16(-32768))

    def body2(_, c):
        lo, hi = c
        mid = (lo >> 1) + (hi >> 1) + ((lo | hi) & 1)
        ge = count_ge16(lo_m, mid.astype(jnp.int16)) >= k2
        return jnp.where(ge, mid, lo), jnp.where(ge, hi, mid - 1)

    c = (i32row(-32768), i32row(32767))
    for _ in range(16):
        c = body2(None, c)
    t_lo = c[0]

    thresh = (t_hi << 16) + (t_lo + 32768)
    tb = jnp.where(thresh >= 0, thresh, int_min - thresh)
    thresh_f = jax.lax.bitcast_convert_type(tb, jnp.float32)
    mask = st >= thresh_f

    m = jnp.max(st, axis=0, keepdims=True)
    p = jnp.where(mask, jnp.exp(st - m), 0.0)
    denom = jnp.sum(p, axis=0, keepdims=True)
    out = jax.lax.dot_general(
        p, v, (((0,), (0,)), ((), ())), preferred_element_type=jnp.float32
    )
    inv_d = (1.0 / denom).reshape(qb, 1)
    o_ref[...] = out * inv_d


@jax.jit
def kernel(query, key, value):
    B, S, HD = query.shape
    H, D = _NUM_HEADS, _HEAD_SIZE
    assert B == 1 and HD == H * D
    k_keep = max(1, int(_TOPK_RATIO * S))
    scale = 1.0 / math.sqrt(D)

    q2 = query.reshape(S, HD)
    k2 = key.reshape(S, HD)
    v2 = value.reshape(S, HD)

    QB = 512
    grid = (H, S // QB)

    out = pl.pallas_call(
        functools.partial(_attn_kernel, k_keep=k_keep, scale=scale),
        grid=grid,
        in_specs=[
            pl.BlockSpec((QB, D), lambda h, qi: (qi, h)),
            pl.BlockSpec((S, D), lambda h, qi: (0, h)),
            pl.BlockSpec((S, D), lambda h, qi: (0, h)),
        ],
        out_specs=pl.BlockSpec((QB, D), lambda h, qi: (qi, h)),
        out_shape=jax.ShapeDtypeStruct((S, HD), jnp.float32),
        compiler_params=pltpu.CompilerParams(
            dimension_semantics=("parallel", "parallel"),
        ),
    )(q2, k2, v2)

    return out.reshape(B, S, HD)


# speedup vs baseline: 1.2177x; 1.0212x over previous
import functools
import math

import jax
import jax.numpy as jnp
from jax.experimental import pallas as pl
from jax.experimental.pallas import tpu as pltpu

_NUM_HEADS = 16
_HEAD_SIZE = 128
_TOPK_RATIO = 0.1

def _attn_kernel(q_ref, k_ref, v_ref, o_ref, *, k_keep, scale):
    q = q_ref[...]
    k = k_ref[...]
    v = v_ref[...]

    st = jax.lax.dot_general(
        k, q * scale, (((1,), (1,)), ((), ())),
        preferred_element_type=jnp.float32,
    )

    b = jax.lax.bitcast_convert_type(st, jnp.int32)
    int_min = jnp.int32(-(2**31))
    keys = jnp.where(b >= 0, b, int_min - b)

    hi16 = (keys >> 16).astype(jnp.int16)
    lo16 = ((keys & 0xFFFF) - 32768).astype(jnp.int16)

    def count_ge16(x16, m16):
        h = x16.shape[0]
        acc = None
        for j in range(0, h, 128):
            ind = jnp.where(x16[j : j + 128] >= m16, jnp.int16(1), jnp.int16(0))
            acc = ind if acc is None else acc + ind
        while acc.shape[0] > 8:
            half = acc.shape[0] // 2
            acc = acc[:half] + acc[half:]
        return jnp.sum(acc.astype(jnp.int32), axis=0, keepdims=True)

    def body1(_, c):
        lo, hi = c
        mid = (lo >> 1) + (hi >> 1) + ((lo | hi) & 1)
        ge = count_ge16(hi16, mid.astype(jnp.int16)) >= k_keep
        return jnp.where(ge, mid, lo), jnp.where(ge, hi, mid - 1)

    qb = st.shape[1]
    i32row = lambda val: jnp.full((1, qb), val, dtype=jnp.int32)
    c = (i32row(-32768), i32row(32767))
    for _ in range(16):
        c = body1(None, c)
    t_hi = c[0]
    t_hi16 = t_hi.astype(jnp.int16)

    c_gt = jnp.where(
        t_hi >= 32767, 0, count_ge16(hi16, (t_hi + 1).astype(jnp.int16))
    )
    k2 = k_keep - c_gt
    lo_m = jnp.where(hi16 == t_hi16, lo16, jnp.int16(-32768))

    def body2(_, c):
        lo, hi = c
        mid = (lo >> 1) + (hi >> 1) + ((lo | hi) & 1)
        ge = count_ge16(lo_m, mid.astype(jnp.int16)) >= k2
        return jnp.where(ge, mid, lo), jnp.where(ge, hi, mid - 1)

    c = (i32row(-32768), i32row(32767))
    for _ in range(16):
        c = body2(None, c)
    t_lo = c[0]

    thresh = (t_hi << 16) + (t_lo + 32768)
    tb = jnp.where(thresh >= 0, thresh, int_min - thresh)
    thresh_f = jax.lax.bitcast_convert_type(tb, jnp.float32)
    mask = st >= thresh_f

    m = jnp.max(st, axis=0, keepdims=True)
    p = jnp.where(mask, jnp.exp(st - m), 0.0)
    denom = jnp.sum(p, axis=0, keepdims=True)
    out = jax.lax.dot_general(
        p, v, (((0,), (0,)), ((), ())), preferred_element_type=jnp.float32
    )
    inv_d = (1.0 / denom).reshape(qb, 1)
    o_ref[...] = out * inv_d


@jax.jit
def kernel(query, key, value):
    B, S, HD = query.shape
    H, D = _NUM_HEADS, _HEAD_SIZE
    assert B == 1 and HD == H * D
    k_keep = max(1, int(_TOPK_RATIO * S))
    scale = 1.0 / math.sqrt(D)

    q2 = query.reshape(S, HD)
    k2 = key.reshape(S, HD)
    v2 = value.reshape(S, HD)

    QB = 512
    grid = (H, S // QB)

    out = pl.pallas_call(
        functools.partial(_attn_kernel, k_keep=k_keep, scale=scale),
        grid=grid,
        in_specs=[
            pl.BlockSpec((QB, D), lambda h, qi: (qi, h)),
            pl.BlockSpec((S, D), lambda h, qi: (0, h)),
            pl.BlockSpec((S, D), lambda h, qi: (0, h)),
        ],
        out_specs=pl.BlockSpec((QB, D), lambda h, qi: (qi, h)),
        out_shape=jax.ShapeDtypeStruct((S, HD), jnp.float32),
        compiler_params=pltpu.CompilerParams(
            dimension_semantics=("parallel", "parallel"),
        ),
    )(q2, k2, v2)

    return out.reshape(B, S, HD)
